# baseline (speedup 1.0000x reference)
import jax
import jax.numpy as jnp
import numpy as np
from jax.experimental import pallas as pl
from jax.experimental.pallas import tpu as pltpu

SEQ_LEN = 512
PRED_LEN = 96
C = 64
K = 8
STATE_NUM = 8
B = 16
F_DIM = PRED_LEN // 2 + 1
OUT_DIM = F_DIM * (2 * K + 1)
SEQT = SEQ_LEN + PRED_LEN
NLAG = SEQ_LEN - 2
NF = SEQ_LEN // 2


def _dft_constants():
    s = np.arange(SEQ_LEN)[:, None]
    f = np.arange(NF)[None, :]
    ang = 2.0 * np.pi * s * f / SEQ_LEN
    fre = np.cos(ang)
    fim = -np.sin(ang)
    w = np.full((NF,), 2.0)
    w[0] = 1.0
    t = np.arange(SEQ_LEN)[None, :]
    fa = np.arange(NF)[:, None]
    ang2 = 2.0 * np.pi * fa * t / SEQ_LEN
    inv_a = w[:, None] * np.cos(ang2) / SEQ_LEN
    inv_b = -w[:, None] * np.sin(ang2) / SEQ_LEN
    s9 = np.arange(PRED_LEN)[:, None]
    f9 = np.arange(F_DIM)[None, :]
    ang9 = 2.0 * np.pi * s9 * f9 / PRED_LEN
    f96re = np.cos(ang9)
    f96im = -np.sin(ang9)
    w9 = np.full((F_DIM,), 2.0)
    w9[0] = 1.0
    w9[-1] = 1.0
    fg = np.arange(F_DIM)[:, None]
    tg = np.arange(PRED_LEN)[None, :]
    angg = 2.0 * np.pi * fg * tg / PRED_LEN
    g_re = w9[:, None] * np.cos(angg) / PRED_LEN
    g_im = -w9[:, None] * np.sin(angg) / PRED_LEN
    c = lambda a: np.asarray(a, np.float32)
    return c(fre), c(fim), c(inv_a), c(inv_b), c(f96re), c(f96im), c(g_re), c(g_im)


_HI = jax.lax.Precision.HIGHEST


def _dot(a, b):
    return jax.lax.dot_general(a, b, (((a.ndim - 1,), (0,)), ((), ())),
                               precision=_HI, preferred_element_type=jnp.float32)


def _lift_kernel(x_ref, y_ref, temp_ref, cwt_ref, bs_ref, fb_ref, mhw_ref,
                 mhb_ref, m1_ref, bc_ref, fre_ref, fim_ref, inva_ref, invb_ref,
                 f96_ref, gre_ref, gim_ref, out_ref):
    x = x_ref[0]
    yh = y_ref[0]

    mu = jnp.mean(x, axis=-1, keepdims=True)
    xc = x - mu
    std = jnp.sqrt(jnp.mean(xc * xc, axis=-1, keepdims=True) + 1e-8)
    xn = xc / std
    yn = (yh - mu) / std

    s_iota = jax.lax.broadcasted_iota(jnp.int32, (1, SEQ_LEN), 1)
    alt_s = jnp.where((s_iota & 1) == 0, 1.0, -1.0)
    rr = _dot(xn, fre_ref[...])
    ri = _dot(xn, fim_ref[...])
    ny = jnp.sum(xn * alt_s, axis=-1, keepdims=True)

    NB = 8
    NBLK = C // NB
    pr_l, pi_l, pn_l = [], [], []
    for a in range(NBLK):
        rr_i, ri_i, ny_i = rr[NB * a:NB * (a + 1)], ri[NB * a:NB * (a + 1)], ny[NB * a:NB * (a + 1)]
        rr_j, ri_j, ny_j = rr[NB * a:], ri[NB * a:], ny[NB * a:]
        w = C - NB * a
        pr_l.append((rr_i[:, None, :] * rr_j[None, :, :] +
                     ri_i[:, None, :] * ri_j[None, :, :]).reshape(NB * w, NF))
        pi_l.append((ri_i[:, None, :] * rr_j[None, :, :] -
                     rr_i[:, None, :] * ri_j[None, :, :]).reshape(NB * w, NF))
        pn_l.append((ny_i[:, None, :] * ny_j[None, :, :]).reshape(NB * w, 1))
    p_re = jnp.concatenate(pr_l, axis=0)
    p_im = jnp.concatenate(pi_l, axis=0)
    p_ny = jnp.concatenate(pn_l, axis=0)
    alt_t = jnp.where((s_iota & 1) == 0, 1.0 / SEQ_LEN, -1.0 / SEQ_LEN)
    cc = (_dot(p_re, inva_ref[...]) + _dot(p_im, invb_ref[...])
          + p_ny * alt_t)

    cc_blk, off = [], 0
    for a in range(NBLK):
        w = C - NB * a
        cc_blk.append(cc[off:off + NB * w].reshape(NB, w, SEQ_LEN))
        off += NB * w

    def _fwd_reduce(ccx):
        n0, n1, _ = ccx.shape
        ca = jnp.abs(ccx)
        ca_int = ca[:, :, 1:-1]
        m = (ca_int >= ca[:, :, :-2]) & (ca_int >= ca[:, :, 2:])
        cam = jnp.where(m, ca_int, 0.0)
        camax = jnp.max(cam, axis=-1)
        lag_iota = jax.lax.broadcasted_iota(jnp.int32, (n0, n1, NLAG), 2)
        code = lag_iota * 2 + (ccx[:, :, 1:-1] < 0.0).astype(jnp.int32)
        hit = cam >= camax[:, :, None]
        sh2 = jnp.min(jnp.where(hit, code, 2 * NLAG), axis=-1)
        r = jnp.where(sh2 & 1 != 0, -camax, camax) * (1.0 / SEQ_LEN)
        return ca, camax, (sh2 >> 1) + 1, r

    def _rev_reduce(ccx, cax):
        n0, n1, _ = ccx.shape
        ca_rn = jnp.concatenate([cax[:, :, 3:], cax[:, :, :1]], axis=-1)
        m2 = (cax[:, :, 2:] >= cax[:, :, 1:-1]) & (cax[:, :, 2:] >= ca_rn)
        cam2 = jnp.where(m2, cax[:, :, 2:], 0.0)
        camax = jnp.max(cam2, axis=-1)
        lag2 = jax.lax.broadcasted_iota(jnp.int32, (n0, n1, NLAG), 2)
        code2 = lag2 * 2 + (ccx[:, :, 2:] < 0.0).astype(jnp.int32)
        hit2 = cam2 >= camax[:, :, None]
        tsel2 = jnp.max(jnp.where(hit2, code2, -1), axis=-1)
        r = jnp.where(tsel2 & 1 != 0, -camax, camax) * (1.0 / SEQ_LEN)
        return camax, SEQ_LEN - ((tsel2 >> 1) + 2), r

    fwd = [_fwd_reduce(cb) for cb in cc_blk]

    cam_rows, sh_rows, r_rows = [], [], []
    for a in range(NBLK):
        cam_p, sh_p, r_p = [], [], []
        for c in range(a):
            lo = NB * (a - c)
            sub_cc = cc_blk[c][:, lo:lo + NB, :]
            sub_ca = fwd[c][0][:, lo:lo + NB, :]
            cmx, shv, rv = _rev_reduce(sub_cc, sub_ca)
            cam_p.append(cmx.T)
            sh_p.append(shv.T)
            r_p.append(rv.T)
        cam_p.append(fwd[a][1])
        sh_p.append(fwd[a][2])
        r_p.append(fwd[a][3])
        cam_rows.append(jnp.concatenate(cam_p, axis=1))
        sh_rows.append(jnp.concatenate(sh_p, axis=1))
        r_rows.append(jnp.concatenate(r_p, axis=1))
    camax = jnp.concatenate(cam_rows, axis=0)
    shift = jnp.concatenate(sh_rows, axis=0)
    r_val = jnp.concatenate(r_rows, axis=0)

    j_iota = jax.lax.broadcasted_iota(jnp.int32, (C, C), 1)
    cur = camax
    lead_l, shift_l, r_l = [], [], []
    for _ in range(K):
        mx = jnp.max(cur, axis=-1, keepdims=True)
        idx = jnp.min(jnp.where(cur >= mx, j_iota, C), axis=-1, keepdims=True)
        sel = j_iota == idx
        lead_l.append(idx)
        shift_l.append(jnp.sum(jnp.where(sel, shift, 0), axis=-1, keepdims=True))
        r_l.append(jnp.sum(jnp.where(sel, r_val, 0.0), axis=-1, keepdims=True))
        cur = jnp.where(sel, -1.0, cur)

    seq = jnp.concatenate([xn, yn], axis=-1)
    onehot = jnp.concatenate(
        [(lead_l[k] == j_iota).astype(jnp.float32) for k in range(K)], axis=0)
    rows = _dot(onehot, seq)
    shifts_km = jnp.concatenate(shift_l, axis=0)
    r_km = jnp.concatenate(r_l, axis=0)
    rows = rows * jnp.sign(r_km)
    for bit in range(9):
        amt = 1 << bit
        rolled = pltpu.roll(rows, amt, 1)
        rows = jnp.where((shifts_km & amt) != 0, rolled, rows)
    win = rows[:, SEQ_LEN:]

    t_inv = 1.0 / temp_ref[0, 0]
    r_abs = jnp.abs(jnp.concatenate(r_l, axis=1))
    z = jnp.concatenate([jnp.ones((C, 1), jnp.float32), r_abs], axis=1) * t_inv
    z = z - jnp.max(z, axis=-1, keepdims=True)
    ez = jnp.exp(z)
    sm = ez / jnp.sum(ez, axis=-1, keepdims=True)
    cf = sm[:, 1:]

    logits = fb_ref[...] + bs_ref[...] + _dot(x, cwt_ref[...])
    logits = logits - jnp.max(logits, axis=-1, keepdims=True)
    el = jnp.exp(logits)
    p = el / jnp.sum(el, axis=-1, keepdims=True)

    q = (p[:, :, None] * cf[:, None, :]).reshape(C, STATE_NUM * K)
    filt = _dot(q, mhw_ref[...]) + _dot(p, mhb_ref[...])

    yf = _dot(yn, f96_ref[...])
    yf_re = yf[:, :F_DIM]
    yf_im = yf[:, F_DIM:]
    sf = _dot(win, f96_ref[...])
    ss_re = jnp.zeros((C, F_DIM), jnp.float32)
    ss_im = jnp.zeros((C, F_DIM), jnp.float32)
    sd_re = jnp.zeros((C, F_DIM), jnp.float32)
    sd_im = jnp.zeros((C, F_DIM), jnp.float32)
    for k in range(K):
        sf_re = sf[k * C:(k + 1) * C, :F_DIM]
        sf_im = sf[k * C:(k + 1) * C, F_DIM:]
        f1 = filt[:, k * F_DIM:(k + 1) * F_DIM]
        f2 = filt[:, (K + k) * F_DIM:(K + k + 1) * F_DIM]
        a_re = sf_re * f1
        a_im = sf_im * f1
        ss_re = ss_re + a_re
        ss_im = ss_im + a_im
        sd_re = sd_re + (a_re - yf_re) * f2
        sd_im = sd_im + (a_im - yf_im) * f2
    f_last = filt[:, 2 * K * F_DIM:]
    y2_re = yf_re * f_last
    y2_im = yf_im * f_last
    mix_re = jnp.concatenate([ss_re, sd_re, y2_re], axis=1)
    mix_im = jnp.concatenate([ss_im, sd_im, y2_im], axis=1)
    out_cat = _dot(jnp.concatenate([mix_re, mix_im], axis=1), m1_ref[...])
    out_cat = out_cat + bc_ref[...]
    y_add = (_dot(out_cat[:, :F_DIM], gre_ref[...]) +
             _dot(out_cat[:, F_DIM:], gim_ref[...]))

    out_ref[0] = (yn + y_add) * std + mu


@jax.jit
def kernel(x, y_hat, temperature, classifier_w, basic_state, factory_bias,
           mix_head_w, mix_head_b, mix_w_real, mix_w_imag, mix_b_real,
           mix_b_imag):
    fre, fim, inva, invb, f96re, f96im, gre, gim = _dft_constants()
    f96 = np.concatenate([f96re, f96im], axis=1)
    mhw_r = mix_head_w.reshape(STATE_NUM * K, OUT_DIM)
    wr_t = mix_w_real.T
    wi_t = mix_w_imag.T
    m1 = jnp.concatenate([jnp.concatenate([wr_t, wi_t], axis=1),
                          jnp.concatenate([-wi_t, wr_t], axis=1)], axis=0)
    bc = jnp.concatenate([mix_b_real, mix_b_imag])[None, :]
    temp2 = temperature.reshape(1, 1)
    fb2 = factory_bias[None, :]
    cwt = classifier_w.T

    full = lambda shape: pl.BlockSpec(shape, lambda b: (0,) * len(shape))
    return pl.pallas_call(
        _lift_kernel,
        grid=(B,),
        in_specs=[
            pl.BlockSpec((1, C, SEQ_LEN), lambda b: (b, 0, 0)),
            pl.BlockSpec((1, C, PRED_LEN), lambda b: (b, 0, 0)),
            full((1, 1)),
            full((SEQ_LEN, STATE_NUM)),
            full((C, STATE_NUM)),
            full((1, STATE_NUM)),
            full((STATE_NUM * K, OUT_DIM)),
            full((STATE_NUM, OUT_DIM)),
            full((2 * 3 * F_DIM, 2 * F_DIM)),
            full((1, 2 * F_DIM)),
            full((SEQ_LEN, NF)),
            full((SEQ_LEN, NF)),
            full((NF, SEQ_LEN)),
            full((NF, SEQ_LEN)),
            full((PRED_LEN, 2 * F_DIM)),
            full((F_DIM, PRED_LEN)),
            full((F_DIM, PRED_LEN)),
        ],
        out_specs=pl.BlockSpec((1, C, PRED_LEN), lambda b: (b, 0, 0)),
        out_shape=jax.ShapeDtypeStruct((B, C, PRED_LEN), jnp.float32),
        compiler_params=pltpu.CompilerParams(
            dimension_semantics=("parallel",)),
    )(x, y_hat, temp2, cwt, basic_state, fb2, mhw_r, mix_head_b, m1, bc,
      fre, fim, inva, invb, f96, gre, gim)


# speedup vs baseline: 1.0867x; 1.0867x over previous
import jax
import jax.numpy as jnp
import numpy as np
from jax.experimental import pallas as pl
from jax.experimental.pallas import tpu as pltpu

SEQ_LEN = 512
PRED_LEN = 96
C = 64
K = 8
STATE_NUM = 8
B = 16
F_DIM = PRED_LEN // 2 + 1
OUT_DIM = F_DIM * (2 * K + 1)
SEQT = SEQ_LEN + PRED_LEN
NLAG = SEQ_LEN - 2
NF = SEQ_LEN // 2


def _dft_constants():
    s = np.arange(SEQ_LEN)[:, None]
    f = np.arange(NF)[None, :]
    ang = 2.0 * np.pi * s * f / SEQ_LEN
    fre = np.cos(ang)
    fim = -np.sin(ang)
    w = np.full((NF,), 2.0)
    w[0] = 1.0
    t = np.arange(SEQ_LEN)[None, :]
    fa = np.arange(NF)[:, None]
    ang2 = 2.0 * np.pi * fa * t / SEQ_LEN
    inv_a = w[:, None] * np.cos(ang2) / SEQ_LEN
    inv_b = -w[:, None] * np.sin(ang2) / SEQ_LEN
    s9 = np.arange(PRED_LEN)[:, None]
    f9 = np.arange(F_DIM)[None, :]
    ang9 = 2.0 * np.pi * s9 * f9 / PRED_LEN
    f96re = np.cos(ang9)
    f96im = -np.sin(ang9)
    w9 = np.full((F_DIM,), 2.0)
    w9[0] = 1.0
    w9[-1] = 1.0
    fg = np.arange(F_DIM)[:, None]
    tg = np.arange(PRED_LEN)[None, :]
    angg = 2.0 * np.pi * fg * tg / PRED_LEN
    g_re = w9[:, None] * np.cos(angg) / PRED_LEN
    g_im = -w9[:, None] * np.sin(angg) / PRED_LEN
    c = lambda a: np.asarray(a, np.float32)
    return c(fre), c(fim), c(inv_a), c(inv_b), c(f96re), c(f96im), c(g_re), c(g_im)


_HI = jax.lax.Precision.HIGHEST


def _dot(a, b):
    return jax.lax.dot_general(a, b, (((a.ndim - 1,), (0,)), ((), ())),
                               precision=_HI, preferred_element_type=jnp.float32)


def _lift_kernel(x_ref, y_ref, temp_ref, cwt_ref, bs_ref, fb_ref, mhw_ref,
                 mhb_ref, m1_ref, bc_ref, fre_ref, fim_ref, inva_ref, invb_ref,
                 f96_ref, gre_ref, gim_ref, out_ref):
    x = x_ref[0]
    yh = y_ref[0]

    mu = jnp.mean(x, axis=-1, keepdims=True)
    xc = x - mu
    std = jnp.sqrt(jnp.mean(xc * xc, axis=-1, keepdims=True) + 1e-8)
    xn = xc / std
    yn = (yh - mu) / std

    s_iota = jax.lax.broadcasted_iota(jnp.int32, (1, SEQ_LEN), 1)
    alt_s = jnp.where((s_iota & 1) == 0, 1.0, -1.0)
    rr = _dot(xn, fre_ref[...])
    ri = _dot(xn, fim_ref[...])
    ny = jnp.sum(xn * alt_s, axis=-1, keepdims=True)

    NB = 16
    NBLK = C // NB
    pr_l, pi_l, pn_l = [], [], []
    for a in range(NBLK):
        rr_i, ri_i, ny_i = rr[NB * a:NB * (a + 1)], ri[NB * a:NB * (a + 1)], ny[NB * a:NB * (a + 1)]
        rr_j, ri_j, ny_j = rr[NB * a:], ri[NB * a:], ny[NB * a:]
        w = C - NB * a
        pr_l.append((rr_i[:, None, :] * rr_j[None, :, :] +
                     ri_i[:, None, :] * ri_j[None, :, :]).reshape(NB * w, NF))
        pi_l.append((ri_i[:, None, :] * rr_j[None, :, :] -
                     rr_i[:, None, :] * ri_j[None, :, :]).reshape(NB * w, NF))
        pn_l.append((ny_i[:, None, :] * ny_j[None, :, :]).reshape(NB * w, 1))
    p_re = jnp.concatenate(pr_l, axis=0)
    p_im = jnp.concatenate(pi_l, axis=0)
    p_ny = jnp.concatenate(pn_l, axis=0)
    alt_t = jnp.where((s_iota & 1) == 0, 1.0 / SEQ_LEN, -1.0 / SEQ_LEN)
    cc = (_dot(p_re, inva_ref[...]) + _dot(p_im, invb_ref[...])
          + p_ny * alt_t)

    cc_blk, off = [], 0
    for a in range(NBLK):
        w = C - NB * a
        cc_blk.append(cc[off:off + NB * w].reshape(NB, w, SEQ_LEN))
        off += NB * w

    def _fwd_reduce(ccx):
        n0, n1, _ = ccx.shape
        ca = jnp.abs(ccx)
        ca_int = ca[:, :, 1:-1]
        m = (ca_int >= ca[:, :, :-2]) & (ca_int >= ca[:, :, 2:])
        cam = jnp.where(m, ca_int, 0.0)
        camax = jnp.max(cam, axis=-1)
        lag_iota = jax.lax.broadcasted_iota(jnp.int32, (n0, n1, NLAG), 2)
        code = lag_iota * 2 + (ccx[:, :, 1:-1] < 0.0).astype(jnp.int32)
        hit = cam >= camax[:, :, None]
        sh2 = jnp.min(jnp.where(hit, code, 2 * NLAG), axis=-1)
        r = jnp.where(sh2 & 1 != 0, -camax, camax) * (1.0 / SEQ_LEN)
        return ca, camax, (sh2 >> 1) + 1, r

    def _rev_reduce(ccx, cax):
        n0, n1, _ = ccx.shape
        ca_rn = jnp.concatenate([cax[:, :, 3:], cax[:, :, :1]], axis=-1)
        m2 = (cax[:, :, 2:] >= cax[:, :, 1:-1]) & (cax[:, :, 2:] >= ca_rn)
        cam2 = jnp.where(m2, cax[:, :, 2:], 0.0)
        camax = jnp.max(cam2, axis=-1)
        lag2 = jax.lax.broadcasted_iota(jnp.int32, (n0, n1, NLAG), 2)
        code2 = lag2 * 2 + (ccx[:, :, 2:] < 0.0).astype(jnp.int32)
        hit2 = cam2 >= camax[:, :, None]
        tsel2 = jnp.max(jnp.where(hit2, code2, -1), axis=-1)
        r = jnp.where(tsel2 & 1 != 0, -camax, camax) * (1.0 / SEQ_LEN)
        return camax, SEQ_LEN - ((tsel2 >> 1) + 2), r

    fwd = [_fwd_reduce(cb) for cb in cc_blk]

    cam_rows, sh_rows, r_rows = [], [], []
    for a in range(NBLK):
        cam_p, sh_p, r_p = [], [], []
        for c in range(a):
            lo = NB * (a - c)
            sub_cc = cc_blk[c][:, lo:lo + NB, :]
            sub_ca = fwd[c][0][:, lo:lo + NB, :]
            cmx, shv, rv = _rev_reduce(sub_cc, sub_ca)
            cam_p.append(cmx.T)
            sh_p.append(shv.T)
            r_p.append(rv.T)
        cam_p.append(fwd[a][1])
        sh_p.append(fwd[a][2])
        r_p.append(fwd[a][3])
        cam_rows.append(jnp.concatenate(cam_p, axis=1))
        sh_rows.append(jnp.concatenate(sh_p, axis=1))
        r_rows.append(jnp.concatenate(r_p, axis=1))
    camax = jnp.concatenate(cam_rows, axis=0)
    shift = jnp.concatenate(sh_rows, axis=0)
    r_val = jnp.concatenate(r_rows, axis=0)

    j_iota = jax.lax.broadcasted_iota(jnp.int32, (C, C), 1)
    cur = camax
    lead_l, shift_l, r_l = [], [], []
    for _ in range(K):
        mx = jnp.max(cur, axis=-1, keepdims=True)
        idx = jnp.min(jnp.where(cur >= mx, j_iota, C), axis=-1, keepdims=True)
        sel = j_iota == idx
        lead_l.append(idx)
        shift_l.append(jnp.sum(jnp.where(sel, shift, 0), axis=-1, keepdims=True))
        r_l.append(jnp.sum(jnp.where(sel, r_val, 0.0), axis=-1, keepdims=True))
        cur = jnp.where(sel, -1.0, cur)

    seq = jnp.concatenate([xn, yn], axis=-1)
    onehot = jnp.concatenate(
        [(lead_l[k] == j_iota).astype(jnp.float32) for k in range(K)], axis=0)
    rows = _dot(onehot, seq)
    shifts_km = jnp.concatenate(shift_l, axis=0)
    r_km = jnp.concatenate(r_l, axis=0)
    rows = rows * jnp.sign(r_km)
    for bit in range(9):
        amt = 1 << bit
        rolled = pltpu.roll(rows, amt, 1)
        rows = jnp.where((shifts_km & amt) != 0, rolled, rows)
    win = rows[:, SEQ_LEN:]

    t_inv = 1.0 / temp_ref[0, 0]
    r_abs = jnp.abs(jnp.concatenate(r_l, axis=1))
    z = jnp.concatenate([jnp.ones((C, 1), jnp.float32), r_abs], axis=1) * t_inv
    z = z - jnp.max(z, axis=-1, keepdims=True)
    ez = jnp.exp(z)
    sm = ez / jnp.sum(ez, axis=-1, keepdims=True)
    cf = sm[:, 1:]

    logits = fb_ref[...] + bs_ref[...] + _dot(x, cwt_ref[...])
    logits = logits - jnp.max(logits, axis=-1, keepdims=True)
    el = jnp.exp(logits)
    p = el / jnp.sum(el, axis=-1, keepdims=True)

    q = (p[:, :, None] * cf[:, None, :]).reshape(C, STATE_NUM * K)
    filt = _dot(q, mhw_ref[...]) + _dot(p, mhb_ref[...])

    yf = _dot(yn, f96_ref[...])
    yf_re = yf[:, :F_DIM]
    yf_im = yf[:, F_DIM:]
    sf = _dot(win, f96_ref[...])
    ss_re = jnp.zeros((C, F_DIM), jnp.float32)
    ss_im = jnp.zeros((C, F_DIM), jnp.float32)
    sd_re = jnp.zeros((C, F_DIM), jnp.float32)
    sd_im = jnp.zeros((C, F_DIM), jnp.float32)
    for k in range(K):
        sf_re = sf[k * C:(k + 1) * C, :F_DIM]
        sf_im = sf[k * C:(k + 1) * C, F_DIM:]
        f1 = filt[:, k * F_DIM:(k + 1) * F_DIM]
        f2 = filt[:, (K + k) * F_DIM:(K + k + 1) * F_DIM]
        a_re = sf_re * f1
        a_im = sf_im * f1
        ss_re = ss_re + a_re
        ss_im = ss_im + a_im
        sd_re = sd_re + (a_re - yf_re) * f2
        sd_im = sd_im + (a_im - yf_im) * f2
    f_last = filt[:, 2 * K * F_DIM:]
    y2_re = yf_re * f_last
    y2_im = yf_im * f_last
    mix_re = jnp.concatenate([ss_re, sd_re, y2_re], axis=1)
    mix_im = jnp.concatenate([ss_im, sd_im, y2_im], axis=1)
    out_cat = _dot(jnp.concatenate([mix_re, mix_im], axis=1), m1_ref[...])
    out_cat = out_cat + bc_ref[...]
    y_add = (_dot(out_cat[:, :F_DIM], gre_ref[...]) +
             _dot(out_cat[:, F_DIM:], gim_ref[...]))

    out_ref[0] = (yn + y_add) * std + mu


@jax.jit
def kernel(x, y_hat, temperature, classifier_w, basic_state, factory_bias,
           mix_head_w, mix_head_b, mix_w_real, mix_w_imag, mix_b_real,
           mix_b_imag):
    fre, fim, inva, invb, f96re, f96im, gre, gim = _dft_constants()
    f96 = np.concatenate([f96re, f96im], axis=1)
    mhw_r = mix_head_w.reshape(STATE_NUM * K, OUT_DIM)
    wr_t = mix_w_real.T
    wi_t = mix_w_imag.T
    m1 = jnp.concatenate([jnp.concatenate([wr_t, wi_t], axis=1),
                          jnp.concatenate([-wi_t, wr_t], axis=1)], axis=0)
    bc = jnp.concatenate([mix_b_real, mix_b_imag])[None, :]
    temp2 = temperature.reshape(1, 1)
    fb2 = factory_bias[None, :]
    cwt = classifier_w.T

    full = lambda shape: pl.BlockSpec(shape, lambda b: (0,) * len(shape))
    return pl.pallas_call(
        _lift_kernel,
        grid=(B,),
        in_specs=[
            pl.BlockSpec((1, C, SEQ_LEN), lambda b: (b, 0, 0)),
            pl.BlockSpec((1, C, PRED_LEN), lambda b: (b, 0, 0)),
            full((1, 1)),
            full((SEQ_LEN, STATE_NUM)),
            full((C, STATE_NUM)),
            full((1, STATE_NUM)),
            full((STATE_NUM * K, OUT_DIM)),
            full((STATE_NUM, OUT_DIM)),
            full((2 * 3 * F_DIM, 2 * F_DIM)),
            full((1, 2 * F_DIM)),
            full((SEQ_LEN, NF)),
            full((SEQ_LEN, NF)),
            full((NF, SEQ_LEN)),
            full((NF, SEQ_LEN)),
            full((PRED_LEN, 2 * F_DIM)),
            full((F_DIM, PRED_LEN)),
            full((F_DIM, PRED_LEN)),
        ],
        out_specs=pl.BlockSpec((1, C, PRED_LEN), lambda b: (b, 0, 0)),
        out_shape=jax.ShapeDtypeStruct((B, C, PRED_LEN), jnp.float32),
        compiler_params=pltpu.CompilerParams(
            dimension_semantics=("parallel",)),
    )(x, y_hat, temp2, cwt, basic_state, fb2, mhw_r, mix_head_b, m1, bc,
      fre, fim, inva, invb, f96, gre, gim)


# speedup vs baseline: 1.0886x; 1.0017x over previous
import jax
import jax.numpy as jnp
import numpy as np
from jax.experimental import pallas as pl
from jax.experimental.pallas import tpu as pltpu

SEQ_LEN = 512
PRED_LEN = 96
C = 64
K = 8
STATE_NUM = 8
B = 16
F_DIM = PRED_LEN // 2 + 1
OUT_DIM = F_DIM * (2 * K + 1)
SEQT = SEQ_LEN + PRED_LEN
NLAG = SEQ_LEN - 2
NF = SEQ_LEN // 2


def _dft_constants():
    s = np.arange(SEQ_LEN)[:, None]
    f = np.arange(NF)[None, :]
    ang = 2.0 * np.pi * s * f / SEQ_LEN
    fre = np.cos(ang)
    fim = -np.sin(ang)
    w = np.full((NF,), 2.0)
    w[0] = 1.0
    t = np.arange(SEQ_LEN)[None, :]
    fa = np.arange(NF)[:, None]
    ang2 = 2.0 * np.pi * fa * t / SEQ_LEN
    inv_a = w[:, None] * np.cos(ang2) / SEQ_LEN
    inv_b = -w[:, None] * np.sin(ang2) / SEQ_LEN
    s9 = np.arange(PRED_LEN)[:, None]
    f9 = np.arange(F_DIM)[None, :]
    ang9 = 2.0 * np.pi * s9 * f9 / PRED_LEN
    f96re = np.cos(ang9)
    f96im = -np.sin(ang9)
    w9 = np.full((F_DIM,), 2.0)
    w9[0] = 1.0
    w9[-1] = 1.0
    fg = np.arange(F_DIM)[:, None]
    tg = np.arange(PRED_LEN)[None, :]
    angg = 2.0 * np.pi * fg * tg / PRED_LEN
    g_re = w9[:, None] * np.cos(angg) / PRED_LEN
    g_im = -w9[:, None] * np.sin(angg) / PRED_LEN
    c = lambda a: np.asarray(a, np.float32)
    return c(fre), c(fim), c(inv_a), c(inv_b), c(f96re), c(f96im), c(g_re), c(g_im)


_HI = jax.lax.Precision.HIGHEST


def _dot(a, b):
    return jax.lax.dot_general(a, b, (((a.ndim - 1,), (0,)), ((), ())),
                               precision=_HI, preferred_element_type=jnp.float32)


def _lift_kernel(x_ref, y_ref, temp_ref, cwt_ref, bs_ref, fb_ref, mhw_ref,
                 mhb_ref, m1_ref, bc_ref, fre_ref, fim_ref, inva_ref, invb_ref,
                 f96_ref, gre_ref, gim_ref, out_ref):
    x = x_ref[0]
    yh = y_ref[0]

    mu = jnp.mean(x, axis=-1, keepdims=True)
    xc = x - mu
    std = jnp.sqrt(jnp.mean(xc * xc, axis=-1, keepdims=True) + 1e-8)
    xn = xc / std
    yn = (yh - mu) / std

    s_iota = jax.lax.broadcasted_iota(jnp.int32, (1, SEQ_LEN), 1)
    alt_s = jnp.where((s_iota & 1) == 0, 1.0, -1.0)
    rr = _dot(xn, fre_ref[...])
    ri = _dot(xn, fim_ref[...])
    ny = jnp.sum(xn * alt_s, axis=-1, keepdims=True)

    NB = 16
    NBLK = C // NB
    pr_l, pi_l, pn_l = [], [], []
    for a in range(NBLK):
        rr_i, ri_i, ny_i = rr[NB * a:NB * (a + 1)], ri[NB * a:NB * (a + 1)], ny[NB * a:NB * (a + 1)]
        rr_j, ri_j, ny_j = rr[NB * a:], ri[NB * a:], ny[NB * a:]
        w = C - NB * a
        pr_l.append((rr_i[:, None, :] * rr_j[None, :, :] +
                     ri_i[:, None, :] * ri_j[None, :, :]).reshape(NB * w, NF))
        pi_l.append((ri_i[:, None, :] * rr_j[None, :, :] -
                     rr_i[:, None, :] * ri_j[None, :, :]).reshape(NB * w, NF))
        pn_l.append((ny_i[:, None, :] * ny_j[None, :, :]).reshape(NB * w, 1))
    p_re = jnp.concatenate(pr_l, axis=0)
    p_im = jnp.concatenate(pi_l, axis=0)
    p_ny = jnp.concatenate(pn_l, axis=0)
    alt_t = jnp.where((s_iota & 1) == 0, 1.0 / SEQ_LEN, -1.0 / SEQ_LEN)
    cc = (_dot(p_re, inva_ref[...]) + _dot(p_im, invb_ref[...])
          + p_ny * alt_t)

    cc_blk, off = [], 0
    for a in range(NBLK):
        w = C - NB * a
        cc_blk.append(cc[off:off + NB * w].reshape(NB, w, SEQ_LEN))
        off += NB * w

    def _fwd_reduce(ccx):
        n0, n1, _ = ccx.shape
        ca = jnp.abs(ccx)
        ca_int = ca[:, :, 1:-1]
        m = (ca_int >= ca[:, :, :-2]) & (ca_int >= ca[:, :, 2:])
        cam = jnp.where(m, ca_int, 0.0)
        camax = jnp.max(cam, axis=-1)
        lag_iota = jax.lax.broadcasted_iota(jnp.int32, (n0, n1, NLAG), 2)
        code = lag_iota * 2 + (ccx[:, :, 1:-1] < 0.0).astype(jnp.int32)
        hit = cam >= camax[:, :, None]
        sh2 = jnp.min(jnp.where(hit, code, 2 * NLAG), axis=-1)
        r = jnp.where(sh2 & 1 != 0, -camax, camax) * (1.0 / SEQ_LEN)
        return ca, cam, camax, (sh2 >> 1) + 1, r

    def _rev_reduce(ccx, cax, camx):
        n0, n1, _ = ccx.shape
        last = cax[:, :, -1:]
        new_col = jnp.where((last >= cax[:, :, -2:-1]) & (last >= cax[:, :, :1]),
                            last, 0.0)
        cam2 = jnp.concatenate([camx[:, :, 1:], new_col], axis=-1)
        camax = jnp.max(cam2, axis=-1)
        lag2 = jax.lax.broadcasted_iota(jnp.int32, (n0, n1, NLAG), 2)
        code2 = lag2 * 2 + (ccx[:, :, 2:] < 0.0).astype(jnp.int32)
        hit2 = cam2 >= camax[:, :, None]
        tsel2 = jnp.max(jnp.where(hit2, code2, -1), axis=-1)
        r = jnp.where(tsel2 & 1 != 0, -camax, camax) * (1.0 / SEQ_LEN)
        return camax, SEQ_LEN - ((tsel2 >> 1) + 2), r

    fwd = [_fwd_reduce(cb) for cb in cc_blk]

    cam_rows, sh_rows, r_rows = [], [], []
    for a in range(NBLK):
        cam_p, sh_p, r_p = [], [], []
        for c in range(a):
            lo = NB * (a - c)
            sub_cc = cc_blk[c][:, lo:lo + NB, :]
            sub_ca = fwd[c][0][:, lo:lo + NB, :]
            sub_cam = fwd[c][1][:, lo:lo + NB, :]
            cmx, shv, rv = _rev_reduce(sub_cc, sub_ca, sub_cam)
            cam_p.append(cmx.T)
            sh_p.append(shv.T)
            r_p.append(rv.T)
        cam_p.append(fwd[a][2])
        sh_p.append(fwd[a][3])
        r_p.append(fwd[a][4])
        cam_rows.append(jnp.concatenate(cam_p, axis=1))
        sh_rows.append(jnp.concatenate(sh_p, axis=1))
        r_rows.append(jnp.concatenate(r_p, axis=1))
    camax = jnp.concatenate(cam_rows, axis=0)
    shift = jnp.concatenate(sh_rows, axis=0)
    r_val = jnp.concatenate(r_rows, axis=0)

    j_iota = jax.lax.broadcasted_iota(jnp.int32, (C, C), 1)
    cur = camax
    lead_l, shift_l, r_l = [], [], []
    for _ in range(K):
        mx = jnp.max(cur, axis=-1, keepdims=True)
        idx = jnp.min(jnp.where(cur >= mx, j_iota, C), axis=-1, keepdims=True)
        sel = j_iota == idx
        lead_l.append(idx)
        shift_l.append(jnp.sum(jnp.where(sel, shift, 0), axis=-1, keepdims=True))
        r_l.append(jnp.sum(jnp.where(sel, r_val, 0.0), axis=-1, keepdims=True))
        cur = jnp.where(sel, -1.0, cur)

    seq = jnp.concatenate([xn, yn], axis=-1)
    onehot = jnp.concatenate(
        [(lead_l[k] == j_iota).astype(jnp.float32) for k in range(K)], axis=0)
    rows = _dot(onehot, seq)
    shifts_km = jnp.concatenate(shift_l, axis=0)
    r_km = jnp.concatenate(r_l, axis=0)
    rows = rows * jnp.sign(r_km)
    for bit in range(9):
        amt = 1 << bit
        rolled = pltpu.roll(rows, amt, 1)
        rows = jnp.where((shifts_km & amt) != 0, rolled, rows)
    win = rows[:, SEQ_LEN:]

    t_inv = 1.0 / temp_ref[0, 0]
    r_abs = jnp.abs(jnp.concatenate(r_l, axis=1))
    z = jnp.concatenate([jnp.ones((C, 1), jnp.float32), r_abs], axis=1) * t_inv
    z = z - jnp.max(z, axis=-1, keepdims=True)
    ez = jnp.exp(z)
    sm = ez / jnp.sum(ez, axis=-1, keepdims=True)
    cf = sm[:, 1:]

    logits = fb_ref[...] + bs_ref[...] + _dot(x, cwt_ref[...])
    logits = logits - jnp.max(logits, axis=-1, keepdims=True)
    el = jnp.exp(logits)
    p = el / jnp.sum(el, axis=-1, keepdims=True)

    q = (p[:, :, None] * cf[:, None, :]).reshape(C, STATE_NUM * K)
    filt = _dot(q, mhw_ref[...]) + _dot(p, mhb_ref[...])

    yf = _dot(yn, f96_ref[...])
    yf_re = yf[:, :F_DIM]
    yf_im = yf[:, F_DIM:]
    sf = _dot(win, f96_ref[...])
    ss_re = jnp.zeros((C, F_DIM), jnp.float32)
    ss_im = jnp.zeros((C, F_DIM), jnp.float32)
    sd_re = jnp.zeros((C, F_DIM), jnp.float32)
    sd_im = jnp.zeros((C, F_DIM), jnp.float32)
    for k in range(K):
        sf_re = sf[k * C:(k + 1) * C, :F_DIM]
        sf_im = sf[k * C:(k + 1) * C, F_DIM:]
        f1 = filt[:, k * F_DIM:(k + 1) * F_DIM]
        f2 = filt[:, (K + k) * F_DIM:(K + k + 1) * F_DIM]
        a_re = sf_re * f1
        a_im = sf_im * f1
        ss_re = ss_re + a_re
        ss_im = ss_im + a_im
        sd_re = sd_re + (a_re - yf_re) * f2
        sd_im = sd_im + (a_im - yf_im) * f2
    f_last = filt[:, 2 * K * F_DIM:]
    y2_re = yf_re * f_last
    y2_im = yf_im * f_last
    mix_re = jnp.concatenate([ss_re, sd_re, y2_re], axis=1)
    mix_im = jnp.concatenate([ss_im, sd_im, y2_im], axis=1)
    out_cat = _dot(jnp.concatenate([mix_re, mix_im], axis=1), m1_ref[...])
    out_cat = out_cat + bc_ref[...]
    y_add = (_dot(out_cat[:, :F_DIM], gre_ref[...]) +
             _dot(out_cat[:, F_DIM:], gim_ref[...]))

    out_ref[0] = (yn + y_add) * std + mu


@jax.jit
def kernel(x, y_hat, temperature, classifier_w, basic_state, factory_bias,
           mix_head_w, mix_head_b, mix_w_real, mix_w_imag, mix_b_real,
           mix_b_imag):
    fre, fim, inva, invb, f96re, f96im, gre, gim = _dft_constants()
    f96 = np.concatenate([f96re, f96im], axis=1)
    mhw_r = mix_head_w.reshape(STATE_NUM * K, OUT_DIM)
    wr_t = mix_w_real.T
    wi_t = mix_w_imag.T
    m1 = jnp.concatenate([jnp.concatenate([wr_t, wi_t], axis=1),
                          jnp.concatenate([-wi_t, wr_t], axis=1)], axis=0)
    bc = jnp.concatenate([mix_b_real, mix_b_imag])[None, :]
    temp2 = temperature.reshape(1, 1)
    fb2 = factory_bias[None, :]
    cwt = classifier_w.T

    full = lambda shape: pl.BlockSpec(shape, lambda b: (0,) * len(shape))
    return pl.pallas_call(
        _lift_kernel,
        grid=(B,),
        in_specs=[
            pl.BlockSpec((1, C, SEQ_LEN), lambda b: (b, 0, 0)),
            pl.BlockSpec((1, C, PRED_LEN), lambda b: (b, 0, 0)),
            full((1, 1)),
            full((SEQ_LEN, STATE_NUM)),
            full((C, STATE_NUM)),
            full((1, STATE_NUM)),
            full((STATE_NUM * K, OUT_DIM)),
            full((STATE_NUM, OUT_DIM)),
            full((2 * 3 * F_DIM, 2 * F_DIM)),
            full((1, 2 * F_DIM)),
            full((SEQ_LEN, NF)),
            full((SEQ_LEN, NF)),
            full((NF, SEQ_LEN)),
            full((NF, SEQ_LEN)),
            full((PRED_LEN, 2 * F_DIM)),
            full((F_DIM, PRED_LEN)),
            full((F_DIM, PRED_LEN)),
        ],
        out_specs=pl.BlockSpec((1, C, PRED_LEN), lambda b: (b, 0, 0)),
        out_shape=jax.ShapeDtypeStruct((B, C, PRED_LEN), jnp.float32),
        compiler_params=pltpu.CompilerParams(
            dimension_semantics=("parallel",)),
    )(x, y_hat, temp2, cwt, basic_state, fb2, mhw_r, mix_head_b, m1, bc,
      fre, fim, inva, invb, f96, gre, gim)
